# baseline (speedup 1.0000x reference)
import functools
import math

import jax
import jax.numpy as jnp
from jax import lax
from jax.experimental import pallas as pl
from jax.experimental.pallas import tpu as pltpu
from jax.experimental.pallas import tpu_sc as plsc

D_MODEL = 64
SCALE = math.sqrt(D_MODEL)
NTOK = 1000000
NB, NT = 4096, 200
NC, NS, L = 2, 16, 16
NW = NC * NS
CHUNK = 128

COLS = 512
NBLK = 977
LROWS = NBLK * COLS
CBITS = COLS.bit_length() - 1
MAXBLK = NTOK // COLS

_mesh = plsc.VectorSubcoreMesh(core_axis_name="c", subcore_axis_name="s")


def _k1_body(a_ref, b_ref, o_ref):
    o_ref[:, 0:64] = a_ref[...].T * SCALE
    o_ref[:, 64:128] = b_ref[...].T * SCALE


def _linearize_table(tt):
    return pl.pallas_call(
        _k1_body,
        grid=(NBLK,),
        in_specs=[
            pl.BlockSpec((64, COLS), lambda i: (0, 2 * i)),
            pl.BlockSpec((64, COLS),
                         lambda i: (0, jnp.minimum(2 * i + 1, MAXBLK))),
        ],
        out_specs=pl.BlockSpec((COLS, 128), lambda i: (i, 0)),
        out_shape=jax.ShapeDtypeStruct((LROWS, 128), jnp.float32),
    )(tt, tt)


@functools.partial(
    pl.kernel,
    out_type=jax.ShapeDtypeStruct((NT, 8, NB // CHUNK, 8, CHUNK), jnp.float32),
    mesh=_mesh,
    scratch_types=[
        pltpu.VMEM((NT, CHUNK), jnp.int32),
        pltpu.VMEM((CHUNK, D_MODEL), jnp.float32),
        pltpu.VMEM((CHUNK, D_MODEL), jnp.float32),
        pltpu.VMEM((D_MODEL, CHUNK), jnp.float32),
        pltpu.VMEM((D_MODEL, CHUNK), jnp.float32),
        pltpu.SemaphoreType.DMA,
        pltpu.SemaphoreType.DMA,
        pltpu.SemaphoreType.DMA,
    ],
    compiler_params=pltpu.CompilerParams(
        use_tc_tiling_on_sc=False, needs_layout_passes=False),
)
def _gather_tr(xg_hbm, tab_hbm, out_hbm, idx_v, rows0, rows1, tb0, tb1,
               gsem, ssem0, ssem1):
    wid = lax.axis_index("s") * NC + lax.axis_index("c")
    pltpu.sync_copy(xg_hbm.at[wid], idx_v)

    def xf_row(t, carry):
        for j in range(CHUNK // L):
            sl = pl.ds(j * L, L)
            q = idx_v[t, sl]
            idx_v[t, sl] = (((q >> (CBITS + 1)) << (CBITS + 1))
                            + ((q & (COLS - 1)) << 1) + ((q >> CBITS) & 1))
        return carry

    lax.fori_loop(0, NT, xf_row, 0)

    iota = lax.iota(jnp.int32, L)
    iotas = [iota + j * L for j in range(CHUNK // L)]

    def fire_gather(t, rows):
        pltpu.async_copy(tab_hbm.at[idx_v.at[t]], rows, gsem)

    def drain_gather(rows):
        pltpu.make_async_copy(tab_hbm.at[pl.ds(0, CHUNK)], rows, gsem).wait()

    def transpose(rows, tb):
        @plsc.parallel_loop(0, D_MODEL, unroll=4)
        def _(d):
            dvec = jnp.full((L,), 0, jnp.int32) + d
            for j in range(CHUNK // L):
                col = plsc.load_gather(rows, [iotas[j], dvec])
                tb[d, pl.ds(j * L, L)] = col

    def fire_stores(t, tb, ssem):
        for db in range(8):
            pltpu.async_copy(tb.at[pl.ds(db * 8, 8)],
                             out_hbm.at[t, db, wid], ssem)

    def drain_stores(tb, ssem):
        for db in range(8):
            pltpu.make_async_copy(tb.at[pl.ds(db * 8, 8)],
                                  out_hbm.at[0, db, wid], ssem).wait()

    fire_gather(0, rows0)

    def super2(u, carry):
        t0 = 2 * u
        drain_gather(rows0)
        fire_gather(t0 + 1, rows1)

        @pl.when(u > 0)
        def _():
            drain_stores(tb0, ssem0)

        transpose(rows0, tb0)
        fire_stores(t0, tb0, ssem0)

        drain_gather(rows1)

        @pl.when(u < NT // 2 - 1)
        def _():
            fire_gather(t0 + 2, rows0)

        @pl.when(u > 0)
        def _():
            drain_stores(tb1, ssem1)

        transpose(rows1, tb1)
        fire_stores(t0 + 1, tb1, ssem1)
        return carry

    lax.fori_loop(0, NT // 2, super2, 0)
    drain_stores(tb0, ssem0)
    drain_stores(tb1, ssem1)


def kernel(x, table):
    tt = table.T
    ltab = _linearize_table(tt).reshape(2 * LROWS, D_MODEL)
    xg = x.T.astype(jnp.int32).reshape(NT, NW, CHUNK).transpose(1, 0, 2)
    o5 = _gather_tr(xg, ltab)
    return o5.transpose(2, 4, 0, 1, 3).reshape(NB, NT, D_MODEL)


# speedup vs baseline: 2.3686x; 2.3686x over previous
import functools
import math

import jax
import jax.numpy as jnp
from jax import lax
from jax.experimental import pallas as pl
from jax.experimental.pallas import tpu as pltpu
from jax.experimental.pallas import tpu_sc as plsc

D_MODEL = 64
SCALE = math.sqrt(D_MODEL)
NTOK = 1000000
NB, NT = 4096, 200
NC, NS, L = 2, 16, 16
NW = NC * NS
CHUNK = 128

COLS = 2048
NBLK = 245
LROWS = NBLK * COLS
CBITS = COLS.bit_length() - 1
MAXBLK = NTOK // COLS

_mesh = plsc.VectorSubcoreMesh(core_axis_name="c", subcore_axis_name="s")


def _k1_body(a_ref, b_ref, o_ref):
    o_ref[:, 0:64] = a_ref[...].T * SCALE
    o_ref[:, 64:128] = b_ref[...].T * SCALE


def _linearize_table(tt):
    return pl.pallas_call(
        _k1_body,
        grid=(NBLK,),
        in_specs=[
            pl.BlockSpec((64, COLS), lambda i: (0, 2 * i)),
            pl.BlockSpec((64, COLS),
                         lambda i: (0, jnp.minimum(2 * i + 1, MAXBLK))),
        ],
        out_specs=pl.BlockSpec((COLS, 128), lambda i: (i, 0)),
        out_shape=jax.ShapeDtypeStruct((LROWS, 128), jnp.float32),
    )(tt, tt)


@functools.partial(
    pl.kernel,
    out_type=jax.ShapeDtypeStruct((NT, 8, NB // CHUNK, 8, CHUNK), jnp.float32),
    mesh=_mesh,
    scratch_types=[
        pltpu.VMEM((NT, CHUNK), jnp.int32),
        pltpu.VMEM((CHUNK, D_MODEL), jnp.float32),
        pltpu.VMEM((CHUNK, D_MODEL), jnp.float32),
        pltpu.VMEM((D_MODEL, CHUNK), jnp.float32),
        pltpu.VMEM((D_MODEL, CHUNK), jnp.float32),
        pltpu.SemaphoreType.DMA,
        pltpu.SemaphoreType.DMA,
        pltpu.SemaphoreType.DMA,
    ],
    compiler_params=pltpu.CompilerParams(
        use_tc_tiling_on_sc=False, needs_layout_passes=False),
)
def _gather_tr(xg_hbm, tab_hbm, out_hbm, idx_v, rows0, rows1, tb0, tb1,
               gsem, ssem0, ssem1):
    wid = lax.axis_index("s") * NC + lax.axis_index("c")
    pltpu.sync_copy(xg_hbm.at[wid], idx_v)

    def xf_row(t, carry):
        for j in range(CHUNK // L):
            sl = pl.ds(j * L, L)
            q = idx_v[t, sl]
            idx_v[t, sl] = (((q >> (CBITS + 1)) << (CBITS + 1))
                            + ((q & (COLS - 1)) << 1) + ((q >> CBITS) & 1))
        return carry

    lax.fori_loop(0, NT, xf_row, 0)

    iota = lax.iota(jnp.int32, L)
    perms = [(iota + k) & 15 for k in range(L)]

    def fire_gather(t, rows):
        pltpu.async_copy(tab_hbm.at[idx_v.at[t]], rows, gsem)

    def drain_gather(rows):
        pltpu.make_async_copy(tab_hbm.at[pl.ds(0, CHUNK)], rows, gsem).wait()

    def transpose(rows, tb):
        @plsc.parallel_loop(0, (CHUNK // L) * (D_MODEL // L), unroll=2)
        def _(b):
            r0 = (b >> 2) << 4
            c0 = (b & 3) << 4
            rv = iota + r0
            for k in range(L):
                pv = perms[k] + c0
                val = plsc.load_gather(rows, [rv, pv])
                plsc.store_scatter(tb, [pv, rv], val)

    def fire_stores(t, tb, ssem):
        for db in range(8):
            pltpu.async_copy(tb.at[pl.ds(db * 8, 8)],
                             out_hbm.at[t, db, wid], ssem)

    def drain_stores(tb, ssem):
        for db in range(8):
            pltpu.make_async_copy(tb.at[pl.ds(db * 8, 8)],
                                  out_hbm.at[0, db, wid], ssem).wait()

    fire_gather(0, rows0)

    def super2(u, carry):
        t0 = 2 * u
        drain_gather(rows0)
        fire_gather(t0 + 1, rows1)

        @pl.when(u > 0)
        def _():
            drain_stores(tb0, ssem0)

        transpose(rows0, tb0)
        fire_stores(t0, tb0, ssem0)

        drain_gather(rows1)

        @pl.when(u < NT // 2 - 1)
        def _():
            fire_gather(t0 + 2, rows0)

        @pl.when(u > 0)
        def _():
            drain_stores(tb1, ssem1)

        transpose(rows1, tb1)
        fire_stores(t0 + 1, tb1, ssem1)
        return carry

    lax.fori_loop(0, NT // 2, super2, 0)
    drain_stores(tb0, ssem0)
    drain_stores(tb1, ssem1)


def kernel(x, table):
    tt = table.T
    ltab = _linearize_table(tt).reshape(2 * LROWS, D_MODEL)
    xg = x.T.astype(jnp.int32).reshape(NT, NW, CHUNK).transpose(1, 0, 2)
    o5 = _gather_tr(xg, ltab)
    return o5.transpose(2, 4, 0, 1, 3).reshape(NB, NT, D_MODEL)


# speedup vs baseline: 2.6819x; 1.1323x over previous
import functools
import math

import jax
import jax.numpy as jnp
from jax import lax
from jax.experimental import pallas as pl
from jax.experimental.pallas import tpu as pltpu
from jax.experimental.pallas import tpu_sc as plsc

D_MODEL = 64
SCALE = math.sqrt(D_MODEL)
NTOK = 1000000
NB, NT = 4096, 200
NC, NS, L = 2, 16, 16
NW = NC * NS
CHUNK = 128

COLS = 4096
NBLK = 123
LROWS = NBLK * COLS
CBITS = COLS.bit_length() - 1
MAXBLK = NTOK // COLS

_mesh = plsc.VectorSubcoreMesh(core_axis_name="c", subcore_axis_name="s")


def _k1_body(a_ref, b_ref, o_ref):
    o_ref[:, 0:64] = a_ref[...].T * SCALE
    o_ref[:, 64:128] = b_ref[...].T * SCALE


def _linearize_table(tt):
    return pl.pallas_call(
        _k1_body,
        grid=(NBLK,),
        in_specs=[
            pl.BlockSpec((64, COLS), lambda i: (0, 2 * i)),
            pl.BlockSpec((64, COLS),
                         lambda i: (0, jnp.minimum(2 * i + 1, MAXBLK))),
        ],
        out_specs=pl.BlockSpec((COLS, 128), lambda i: (i, 0)),
        out_shape=jax.ShapeDtypeStruct((LROWS, 128), jnp.float32),
    )(tt, tt)


@functools.partial(
    pl.kernel,
    out_type=jax.ShapeDtypeStruct((NT, 8, NB // CHUNK, 8, CHUNK), jnp.float32),
    mesh=_mesh,
    scratch_types=[
        pltpu.VMEM((NT, CHUNK), jnp.int32),
        pltpu.VMEM((CHUNK, D_MODEL), jnp.float32),
        pltpu.VMEM((CHUNK, D_MODEL), jnp.float32),
        pltpu.VMEM((D_MODEL, CHUNK), jnp.float32),
        pltpu.VMEM((D_MODEL, CHUNK), jnp.float32),
        pltpu.SemaphoreType.DMA,
        pltpu.SemaphoreType.DMA,
        pltpu.SemaphoreType.DMA,
    ],
    compiler_params=pltpu.CompilerParams(
        use_tc_tiling_on_sc=False, needs_layout_passes=False),
)
def _gather_tr(xg_hbm, tab_hbm, out_hbm, idx_v, rows0, rows1, tb0, tb1,
               gsem, ssem0, ssem1):
    wid = lax.axis_index("s") * NC + lax.axis_index("c")
    pltpu.sync_copy(xg_hbm.at[wid], idx_v)

    def xf_row(t, carry):
        for j in range(CHUNK // L):
            sl = pl.ds(j * L, L)
            q = idx_v[t, sl]
            idx_v[t, sl] = (((q >> (CBITS + 1)) << (CBITS + 1))
                            + ((q & (COLS - 1)) << 1) + ((q >> CBITS) & 1))
        return carry

    lax.fori_loop(0, NT, xf_row, 0)

    iota = lax.iota(jnp.int32, L)
    perms = [(iota + k) & 15 for k in range(L)]

    def fire_gather(t, rows):
        pltpu.async_copy(tab_hbm.at[idx_v.at[t]], rows, gsem)

    def drain_gather(rows):
        pltpu.make_async_copy(tab_hbm.at[pl.ds(0, CHUNK)], rows, gsem).wait()

    def transpose(rows, tb):
        @plsc.parallel_loop(0, (CHUNK // L) * (D_MODEL // L), unroll=2)
        def _(b):
            r0 = (b >> 2) << 4
            c0 = (b & 3) << 4
            rv = iota + r0
            for k in range(L):
                pv = perms[k] + c0
                val = plsc.load_gather(rows, [rv, pv])
                plsc.store_scatter(tb, [pv, rv], val)

    def fire_stores(t, tb, ssem):
        for db in range(8):
            pltpu.async_copy(tb.at[pl.ds(db * 8, 8)],
                             out_hbm.at[t, db, wid], ssem)

    def drain_stores(tb, ssem):
        for db in range(8):
            pltpu.make_async_copy(tb.at[pl.ds(db * 8, 8)],
                                  out_hbm.at[0, db, wid], ssem).wait()

    fire_gather(0, rows0)

    def super2(u, carry):
        t0 = 2 * u
        drain_gather(rows0)
        fire_gather(t0 + 1, rows1)

        @pl.when(u > 0)
        def _():
            drain_stores(tb0, ssem0)

        transpose(rows0, tb0)
        fire_stores(t0, tb0, ssem0)

        drain_gather(rows1)

        @pl.when(u < NT // 2 - 1)
        def _():
            fire_gather(t0 + 2, rows0)

        @pl.when(u > 0)
        def _():
            drain_stores(tb1, ssem1)

        transpose(rows1, tb1)
        fire_stores(t0 + 1, tb1, ssem1)
        return carry

    lax.fori_loop(0, NT // 2, super2, 0)
    drain_stores(tb0, ssem0)
    drain_stores(tb1, ssem1)


def kernel(x, table):
    tt = table.T
    ltab = _linearize_table(tt).reshape(2 * LROWS, D_MODEL)
    xg = x.T.astype(jnp.int32).reshape(NT, NW, CHUNK).transpose(1, 0, 2)
    o5 = _gather_tr(xg, ltab)
    return o5.transpose(2, 4, 0, 1, 3).reshape(NB, NT, D_MODEL)


# speedup vs baseline: 2.8610x; 1.0668x over previous
import functools
import math

import jax
import jax.numpy as jnp
from jax import lax
from jax.experimental import pallas as pl
from jax.experimental.pallas import tpu as pltpu
from jax.experimental.pallas import tpu_sc as plsc

D_MODEL = 64
SCALE = math.sqrt(D_MODEL)
NTOK = 1000000
NB, NT = 4096, 200
NC, NS, L = 2, 16, 16
NW = NC * NS
CHUNK = 128

COLS = 8192
NBLK = 62
LROWS = NBLK * COLS
CBITS = COLS.bit_length() - 1
MAXBLK = NTOK // COLS

_mesh = plsc.VectorSubcoreMesh(core_axis_name="c", subcore_axis_name="s")


def _k1_body(a_ref, b_ref, o_ref):
    o_ref[:, 0:64] = a_ref[...].T * SCALE
    o_ref[:, 64:128] = b_ref[...].T * SCALE


def _linearize_table(tt):
    return pl.pallas_call(
        _k1_body,
        grid=(NBLK,),
        in_specs=[
            pl.BlockSpec((64, COLS), lambda i: (0, 2 * i)),
            pl.BlockSpec((64, COLS),
                         lambda i: (0, jnp.minimum(2 * i + 1, MAXBLK))),
        ],
        out_specs=pl.BlockSpec((COLS, 128), lambda i: (i, 0)),
        out_shape=jax.ShapeDtypeStruct((LROWS, 128), jnp.float32),
    )(tt, tt)


@functools.partial(
    pl.kernel,
    out_type=jax.ShapeDtypeStruct((NT, 8, NB // CHUNK, 8, CHUNK), jnp.float32),
    mesh=_mesh,
    scratch_types=[
        pltpu.VMEM((NT, CHUNK), jnp.int32),
        pltpu.VMEM((CHUNK, D_MODEL), jnp.float32),
        pltpu.VMEM((CHUNK, D_MODEL), jnp.float32),
        pltpu.VMEM((D_MODEL, CHUNK), jnp.float32),
        pltpu.VMEM((D_MODEL, CHUNK), jnp.float32),
        pltpu.SemaphoreType.DMA,
        pltpu.SemaphoreType.DMA,
        pltpu.SemaphoreType.DMA,
    ],
    compiler_params=pltpu.CompilerParams(
        use_tc_tiling_on_sc=False, needs_layout_passes=False),
)
def _gather_tr(xg_hbm, tab_hbm, out_hbm, idx_v, rows0, rows1, tb0, tb1,
               gsem, ssem0, ssem1):
    wid = lax.axis_index("s") * NC + lax.axis_index("c")
    pltpu.sync_copy(xg_hbm.at[wid], idx_v)

    def xf_row(t, carry):
        for j in range(CHUNK // L):
            sl = pl.ds(j * L, L)
            q = idx_v[t, sl]
            idx_v[t, sl] = (((q >> (CBITS + 1)) << (CBITS + 1))
                            + ((q & (COLS - 1)) << 1) + ((q >> CBITS) & 1))
        return carry

    lax.fori_loop(0, NT, xf_row, 0)

    iota = lax.iota(jnp.int32, L)
    perms = [(iota + k) & 15 for k in range(L)]

    def fire_gather(t, rows):
        pltpu.async_copy(tab_hbm.at[idx_v.at[t]], rows, gsem)

    def drain_gather(rows):
        pltpu.make_async_copy(tab_hbm.at[pl.ds(0, CHUNK)], rows, gsem).wait()

    def transpose(rows, tb):
        @plsc.parallel_loop(0, (CHUNK // L) * (D_MODEL // L), unroll=2)
        def _(b):
            r0 = (b >> 2) << 4
            c0 = (b & 3) << 4
            rv = iota + r0
            for k in range(L):
                pv = perms[k] + c0
                val = plsc.load_gather(rows, [rv, pv])
                plsc.store_scatter(tb, [pv, rv], val)

    def fire_stores(t, tb, ssem):
        for db in range(8):
            pltpu.async_copy(tb.at[pl.ds(db * 8, 8)],
                             out_hbm.at[t, db, wid], ssem)

    def drain_stores(tb, ssem):
        for db in range(8):
            pltpu.make_async_copy(tb.at[pl.ds(db * 8, 8)],
                                  out_hbm.at[0, db, wid], ssem).wait()

    fire_gather(0, rows0)

    def super2(u, carry):
        t0 = 2 * u
        drain_gather(rows0)
        fire_gather(t0 + 1, rows1)

        @pl.when(u > 0)
        def _():
            drain_stores(tb0, ssem0)

        transpose(rows0, tb0)
        fire_stores(t0, tb0, ssem0)

        drain_gather(rows1)

        @pl.when(u < NT // 2 - 1)
        def _():
            fire_gather(t0 + 2, rows0)

        @pl.when(u > 0)
        def _():
            drain_stores(tb1, ssem1)

        transpose(rows1, tb1)
        fire_stores(t0 + 1, tb1, ssem1)
        return carry

    lax.fori_loop(0, NT // 2, super2, 0)
    drain_stores(tb0, ssem0)
    drain_stores(tb1, ssem1)


def kernel(x, table):
    tt = table.T
    ltab = _linearize_table(tt).reshape(2 * LROWS, D_MODEL)
    xg = x.T.astype(jnp.int32).reshape(NT, NW, CHUNK).transpose(1, 0, 2)
    o5 = _gather_tr(xg, ltab)
    return o5.transpose(2, 4, 0, 1, 3).reshape(NB, NT, D_MODEL)


# speedup vs baseline: 2.9489x; 1.0307x over previous
import functools
import math

import jax
import jax.numpy as jnp
from jax import lax
from jax.experimental import pallas as pl
from jax.experimental.pallas import tpu as pltpu
from jax.experimental.pallas import tpu_sc as plsc

D_MODEL = 64
SCALE = math.sqrt(D_MODEL)
NTOK = 1000000
NB, NT = 4096, 200
NC, NS, L = 2, 16, 16
NW = NC * NS
CHUNK = 128

COLS = 16384
NBLK = 31
LROWS = NBLK * COLS
CBITS = COLS.bit_length() - 1
MAXBLK = NTOK // COLS

_mesh = plsc.VectorSubcoreMesh(core_axis_name="c", subcore_axis_name="s")


def _k1_body(a_ref, b_ref, o_ref):
    o_ref[:, 0:64] = a_ref[...].T * SCALE
    o_ref[:, 64:128] = b_ref[...].T * SCALE


def _linearize_table(tt):
    return pl.pallas_call(
        _k1_body,
        grid=(NBLK,),
        in_specs=[
            pl.BlockSpec((64, COLS), lambda i: (0, 2 * i)),
            pl.BlockSpec((64, COLS),
                         lambda i: (0, jnp.minimum(2 * i + 1, MAXBLK))),
        ],
        out_specs=pl.BlockSpec((COLS, 128), lambda i: (i, 0)),
        out_shape=jax.ShapeDtypeStruct((LROWS, 128), jnp.float32),
    )(tt, tt)


@functools.partial(
    pl.kernel,
    out_type=jax.ShapeDtypeStruct((NT, 8, NB // CHUNK, 8, CHUNK), jnp.float32),
    mesh=_mesh,
    scratch_types=[
        pltpu.VMEM((NT, CHUNK), jnp.int32),
        pltpu.VMEM((CHUNK, D_MODEL), jnp.float32),
        pltpu.VMEM((CHUNK, D_MODEL), jnp.float32),
        pltpu.VMEM((D_MODEL, CHUNK), jnp.float32),
        pltpu.VMEM((D_MODEL, CHUNK), jnp.float32),
        pltpu.SemaphoreType.DMA,
        pltpu.SemaphoreType.DMA,
        pltpu.SemaphoreType.DMA,
    ],
    compiler_params=pltpu.CompilerParams(
        use_tc_tiling_on_sc=False, needs_layout_passes=False),
)
def _gather_tr(xg_hbm, tab_hbm, out_hbm, idx_v, rows0, rows1, tb0, tb1,
               gsem, ssem0, ssem1):
    wid = lax.axis_index("s") * NC + lax.axis_index("c")
    pltpu.sync_copy(xg_hbm.at[wid], idx_v)

    def xf_row(t, carry):
        for j in range(CHUNK // L):
            sl = pl.ds(j * L, L)
            q = idx_v[t, sl]
            idx_v[t, sl] = (((q >> (CBITS + 1)) << (CBITS + 1))
                            + ((q & (COLS - 1)) << 1) + ((q >> CBITS) & 1))
        return carry

    lax.fori_loop(0, NT, xf_row, 0)

    iota = lax.iota(jnp.int32, L)
    perms = [(iota + k) & 15 for k in range(L)]

    def fire_gather(t, rows):
        pltpu.async_copy(tab_hbm.at[idx_v.at[t]], rows, gsem)

    def drain_gather(rows):
        pltpu.make_async_copy(tab_hbm.at[pl.ds(0, CHUNK)], rows, gsem).wait()

    def transpose(rows, tb):
        @plsc.parallel_loop(0, (CHUNK // L) * (D_MODEL // L), unroll=2)
        def _(b):
            r0 = (b >> 2) << 4
            c0 = (b & 3) << 4
            rv = iota + r0
            for k in range(L):
                pv = perms[k] + c0
                val = plsc.load_gather(rows, [rv, pv])
                plsc.store_scatter(tb, [pv, rv], val)

    def fire_stores(t, tb, ssem):
        for db in range(8):
            pltpu.async_copy(tb.at[pl.ds(db * 8, 8)],
                             out_hbm.at[t, db, wid], ssem)

    def drain_stores(tb, ssem):
        for db in range(8):
            pltpu.make_async_copy(tb.at[pl.ds(db * 8, 8)],
                                  out_hbm.at[0, db, wid], ssem).wait()

    fire_gather(0, rows0)

    def super2(u, carry):
        t0 = 2 * u
        drain_gather(rows0)
        fire_gather(t0 + 1, rows1)

        @pl.when(u > 0)
        def _():
            drain_stores(tb0, ssem0)

        transpose(rows0, tb0)
        fire_stores(t0, tb0, ssem0)

        drain_gather(rows1)

        @pl.when(u < NT // 2 - 1)
        def _():
            fire_gather(t0 + 2, rows0)

        @pl.when(u > 0)
        def _():
            drain_stores(tb1, ssem1)

        transpose(rows1, tb1)
        fire_stores(t0 + 1, tb1, ssem1)
        return carry

    lax.fori_loop(0, NT // 2, super2, 0)
    drain_stores(tb0, ssem0)
    drain_stores(tb1, ssem1)


def kernel(x, table):
    tt = table.T
    ltab = _linearize_table(tt).reshape(2 * LROWS, D_MODEL)
    xg = x.T.astype(jnp.int32).reshape(NT, NW, CHUNK).transpose(1, 0, 2)
    o5 = _gather_tr(xg, ltab)
    return o5.transpose(2, 4, 0, 1, 3).reshape(NB, NT, D_MODEL)
